# baseline (speedup 1.0000x reference)
import jax
import jax.numpy as jnp
from jax import lax
from jax.experimental import pallas as pl
from jax.experimental.pallas import tpu as pltpu

_EMBED = 256
_L = 512
_ALPHA = 0.4
_CHAIN = 5
_TOPP = 10
_WINDOWS = (1, 2, 3, 4, 5)
_STRIDES = (1, 1, 2, 2, 3)
_NPOS = tuple((_L - w) // s + 1 for w, s in zip(_WINDOWS, _STRIDES))
_POS_OFF = tuple(sum(_NPOS[:i]) for i in range(len(_NPOS)))
_NPOS_TOT = sum(_NPOS)
_NPOS_PAD = _NPOS_TOT + 1
_NUNITS = len(_WINDOWS) * _TOPP


def _iota(n, m, dim):
    return lax.broadcasted_iota(jnp.int32, (n, m), dim)


def _stage1(tok_ref, tn_ref, cooc_ref, pos_ref):
    t = tok_ref[0]
    tn = tn_ref[0]

    sims = lax.dot_general(tn, tn, (((1,), (1,)), ((), ())),
                           preferred_element_type=jnp.float32)
    row = _iota(_L, _L, 0)
    col = _iota(_L, _L, 1)
    eye = row == col
    s_work = jnp.where(eye, -9.0, sims)

    cooc = t
    for _ in range(1, _CHAIN):
        bv = jnp.max(s_work, axis=1, keepdims=True)
        first = jnp.min(jnp.where(s_work == bv, col, _L), axis=1,
                        keepdims=True)
        onehot = col == first
        stop = bv < _ALPHA
        pickcond = (stop & eye) | (~stop & onehot)
        pickmat = jnp.where(pickcond, 1.0, 0.0)
        picked = lax.dot_general(pickmat, t, (((1,), (0,)), ((), ())),
                                 preferred_element_type=jnp.float32,
                                 precision=lax.Precision.HIGHEST)
        cooc = cooc + picked
        s_work = jnp.where(onehot, -9.0, s_work)
    cooc_ref[0] = cooc * (1.0 / _CHAIN)

    s2 = t[0:511] + t[1:512]
    s3 = s2[0:510] + t[2:512]
    s4 = s3[0:509] + t[3:512]
    s5 = s4[0:508] + t[4:512]
    sums = (t, s2, s3, s4, s5)
    for sc in range(5):
        w, st, n_pos, off = _WINDOWS[sc], _STRIDES[sc], _NPOS[sc], _POS_OFF[sc]
        s_full = sums[sc]
        if st == 1:
            p = s_full[0:n_pos] * (1.0 / w)
        else:
            n_full = s_full.shape[0]
            sel = jnp.where(_iota(n_pos, n_full, 1) == _iota(n_pos, n_full, 0) * st,
                            1.0, 0.0)
            p = lax.dot_general(sel, s_full, (((1,), (0,)), ((), ())),
                                preferred_element_type=jnp.float32,
                                precision=lax.Precision.HIGHEST) * (1.0 / w)
        pos_ref[0, off:off + n_pos, :] = p
    pos_ref[0, _NPOS_TOT:_NPOS_PAD, :] = jnp.zeros((1, _EMBED), jnp.float32)


_BS = 4


def _stage2(npos_ref, ncooc_ref, pos_ref, cooc_ref, w_ref, b_ref, out_ref,
            iou_ref):
  for bb in range(_BS):
    ncooc = ncooc_ref[bb]

    rsel_parts = []
    csel_parts = []
    for sc in range(5):
        n_pos, off = _NPOS[sc], _POS_OFF[sc]
        np_ = npos_ref[bb, off:off + n_pos, :]
        iou = lax.dot_general(np_, ncooc, (((1,), (1,)), ((), ())),
                              preferred_element_type=jnp.float32)
        iou_ref[bb, 0:n_pos, :] = iou
        colc = _iota(n_pos, _L, 1)
        rm = jnp.max(iou, axis=1, keepdims=True)
        ra = jnp.min(jnp.where(iou == rm, colc, _L), axis=1, keepdims=True)
        riota = _iota(n_pos, 1, 0)
        lane = _iota(1, _L, 1)
        hist = []
        for kk in range(_TOPP):
            gv = jnp.max(rm, axis=0, keepdims=True)
            rs = jnp.min(jnp.where(rm == gv, riota, n_pos))
            cs = jnp.min(jnp.where(riota == rs, ra, _L))
            rsel_parts.append(jnp.reshape(rs + off, (1, 1)))
            csel_parts.append(jnp.reshape(cs, (1, 1)))
            irow = iou_ref[bb, pl.ds(rs, 1), :]
            irow = jnp.where(lane == cs, -9.0, irow)
            for rj, cj in hist:
                irow = jnp.where((rj == rs) & (lane == cj), -9.0, irow)
            nm = jnp.max(irow, axis=1, keepdims=True)
            na = jnp.min(jnp.where(irow == nm, lane, _L), axis=1, keepdims=True)
            sel = riota == rs
            rm = jnp.where(sel, nm, rm)
            ra = jnp.where(sel, na, ra)
            hist.append((rs, cs))

    pad = jnp.zeros((1, 1), jnp.int32)
    rsel = jnp.concatenate(rsel_parts + [pad] * 6, axis=0)
    csel = jnp.concatenate(csel_parts + [pad] * 6, axis=0)
    pa = jnp.where(_iota(56, _NPOS_PAD, 1) == rsel, 1.0, 0.0)
    ca = jnp.where(_iota(56, _L, 1) == csel, 1.0, 0.0)
    left = lax.dot_general(pa, pos_ref[bb], (((1,), (0,)), ((), ())),
                           preferred_element_type=jnp.float32,
                           precision=lax.Precision.HIGHEST)
    right = lax.dot_general(ca, cooc_ref[bb], (((1,), (0,)), ((), ())),
                            preferred_element_type=jnp.float32,
                            precision=lax.Precision.HIGHEST)
    pr = jnp.concatenate([left, right], axis=1)
    units = lax.dot_general(pr, w_ref[...], (((1,), (1,)), ((), ())),
                            preferred_element_type=jnp.float32)
    units = units + b_ref[...]
    unorm = jnp.sqrt(jnp.sum(units * units, axis=1, keepdims=True))
    units = units / (unorm + 1e-12)
    out_ref[bb] = units[0:_NUNITS]


def kernel(tokens, W, b):
    B = tokens.shape[0]
    f32 = jnp.float32

    tn = tokens / (jnp.linalg.norm(tokens, axis=-1, keepdims=True) + 1e-12)

    cooc_vec, pos_all = pl.pallas_call(
        _stage1,
        grid=(B,),
        in_specs=[
            pl.BlockSpec((1, _L, _EMBED), lambda i: (i, 0, 0)),
            pl.BlockSpec((1, _L, _EMBED), lambda i: (i, 0, 0)),
        ],
        out_specs=[
            pl.BlockSpec((1, _L, _EMBED), lambda i: (i, 0, 0)),
            pl.BlockSpec((1, _NPOS_PAD, _EMBED), lambda i: (i, 0, 0)),
        ],
        out_shape=[
            jax.ShapeDtypeStruct((B, _L, _EMBED), f32),
            jax.ShapeDtypeStruct((B, _NPOS_PAD, _EMBED), f32),
        ],
        compiler_params=pltpu.CompilerParams(
            dimension_semantics=("arbitrary",),
        ),
    )(tokens, tn)

    ncooc = cooc_vec / (jnp.linalg.norm(cooc_vec, axis=-1, keepdims=True) + 1e-12)
    npos_parts = []
    for off, n_pos in zip(_POS_OFF, _NPOS):
        seg = lax.slice_in_dim(pos_all, off, off + n_pos, axis=1)
        npos_parts.append(seg / (jnp.linalg.norm(seg, axis=-1, keepdims=True) + 1e-12))
    npos_parts.append(jnp.zeros((B, _NPOS_PAD - _NPOS_TOT, _EMBED), f32))
    npos = jnp.concatenate(npos_parts, axis=1)

    out = pl.pallas_call(
        _stage2,
        grid=(B // _BS,),
        in_specs=[
            pl.BlockSpec((_BS, _NPOS_PAD, _EMBED), lambda i: (i, 0, 0)),
            pl.BlockSpec((_BS, _L, _EMBED), lambda i: (i, 0, 0)),
            pl.BlockSpec((_BS, _NPOS_PAD, _EMBED), lambda i: (i, 0, 0)),
            pl.BlockSpec((_BS, _L, _EMBED), lambda i: (i, 0, 0)),
            pl.BlockSpec((_EMBED, 2 * _EMBED), lambda i: (0, 0)),
            pl.BlockSpec((1, _EMBED), lambda i: (0, 0)),
        ],
        out_specs=pl.BlockSpec((_BS, _NUNITS, _EMBED), lambda i: (i, 0, 0)),
        out_shape=jax.ShapeDtypeStruct((B, _NUNITS, _EMBED), f32),
        scratch_shapes=[pltpu.VMEM((_BS, _L, _L), f32)],
        compiler_params=pltpu.CompilerParams(
            dimension_semantics=("arbitrary",),
        ),
    )(npos, ncooc, pos_all, cooc_vec, W, b.reshape(1, _EMBED))
    return out


# speedup vs baseline: 1.1917x; 1.1917x over previous
import jax
import jax.numpy as jnp
from jax import lax
from jax.experimental import pallas as pl
from jax.experimental.pallas import tpu as pltpu

_EMBED = 256
_L = 512
_ALPHA = 0.4
_CHAIN = 5
_TOPP = 10
_WINDOWS = (1, 2, 3, 4, 5)
_STRIDES = (1, 1, 2, 2, 3)
_NPOS = tuple((_L - w) // s + 1 for w, s in zip(_WINDOWS, _STRIDES))
_POS_OFF = tuple(sum(_NPOS[:i]) for i in range(len(_NPOS)))
_NPOS_TOT = sum(_NPOS)
_NPOS_PAD = _NPOS_TOT + 1
_NUNITS = len(_WINDOWS) * _TOPP


def _iota(n, m, dim):
    return lax.broadcasted_iota(jnp.int32, (n, m), dim)


def _stage1(tok_ref, tn_ref, cooc_ref, pos_ref):
    t = tok_ref[0]
    tn = tn_ref[0]

    sims = lax.dot_general(tn, tn, (((1,), (1,)), ((), ())),
                           preferred_element_type=jnp.float32)
    row = _iota(_L, _L, 0)
    col = _iota(_L, _L, 1)
    eye = row == col
    s_work = jnp.where(eye, -9.0, sims)

    cooc = t
    for _ in range(1, _CHAIN):
        bv = jnp.max(s_work, axis=1, keepdims=True)
        first = jnp.min(jnp.where(s_work == bv, col, _L), axis=1,
                        keepdims=True)
        onehot = col == first
        stop = bv < _ALPHA
        pickcond = (stop & eye) | (~stop & onehot)
        pickmat = jnp.where(pickcond, 1.0, 0.0)
        picked = lax.dot_general(pickmat, t, (((1,), (0,)), ((), ())),
                                 preferred_element_type=jnp.float32,
                                 precision=lax.Precision.HIGHEST)
        cooc = cooc + picked
        s_work = jnp.where(onehot, -9.0, s_work)
    cooc_ref[0] = cooc * (1.0 / _CHAIN)

    s2 = t[0:511] + t[1:512]
    s3 = s2[0:510] + t[2:512]
    s4 = s3[0:509] + t[3:512]
    s5 = s4[0:508] + t[4:512]
    sums = (t, s2, s3, s4, s5)
    for sc in range(5):
        w, st, n_pos, off = _WINDOWS[sc], _STRIDES[sc], _NPOS[sc], _POS_OFF[sc]
        s_full = sums[sc]
        if st == 1:
            p = s_full[0:n_pos] * (1.0 / w)
        else:
            n_full = s_full.shape[0]
            sel = jnp.where(_iota(n_pos, n_full, 1) == _iota(n_pos, n_full, 0) * st,
                            1.0, 0.0)
            p = lax.dot_general(sel, s_full, (((1,), (0,)), ((), ())),
                                preferred_element_type=jnp.float32,
                                precision=lax.Precision.HIGHEST) * (1.0 / w)
        pos_ref[0, off:off + n_pos, :] = p
    pos_ref[0, _NPOS_TOT:_NPOS_PAD, :] = jnp.zeros((1, _EMBED), jnp.float32)


_BS = 2


def _stage2(npos_ref, ncooc_ref, pos_ref, cooc_ref, w_ref, b_ref, out_ref,
            iou_ref):
  for bb in range(_BS):
    ncooc = ncooc_ref[bb]

    rsel_parts = []
    csel_parts = []
    for sc in range(5):
        n_pos, off = _NPOS[sc], _POS_OFF[sc]
        np_ = npos_ref[bb, off:off + n_pos, :]
        iou = lax.dot_general(np_, ncooc, (((1,), (1,)), ((), ())),
                              preferred_element_type=jnp.float32)
        iou_ref[bb, 0:n_pos, :] = iou
        colc = _iota(n_pos, _L, 1)
        rm = jnp.max(iou, axis=1, keepdims=True)
        ra = jnp.min(jnp.where(iou == rm, colc, _L), axis=1, keepdims=True)
        riota = _iota(n_pos, 1, 0)
        lane = _iota(1, _L, 1)
        hist = []
        for kk in range(_TOPP):
            gv = jnp.max(rm, axis=0, keepdims=True)
            rs = jnp.min(jnp.where(rm == gv, riota, n_pos))
            cs = jnp.min(jnp.where(riota == rs, ra, _L))
            rsel_parts.append(jnp.reshape(rs + off, (1, 1)))
            csel_parts.append(jnp.reshape(cs, (1, 1)))
            irow = iou_ref[bb, pl.ds(rs, 1), :]
            irow = jnp.where(lane == cs, -9.0, irow)
            for rj, cj in hist:
                irow = jnp.where((rj == rs) & (lane == cj), -9.0, irow)
            nm = jnp.max(irow, axis=1, keepdims=True)
            na = jnp.min(jnp.where(irow == nm, lane, _L), axis=1, keepdims=True)
            sel = riota == rs
            rm = jnp.where(sel, nm, rm)
            ra = jnp.where(sel, na, ra)
            hist.append((rs, cs))

    pad = jnp.zeros((1, 1), jnp.int32)
    rsel = jnp.concatenate(rsel_parts + [pad] * 6, axis=0)
    csel = jnp.concatenate(csel_parts + [pad] * 6, axis=0)
    pa = jnp.where(_iota(56, _NPOS_PAD, 1) == rsel, 1.0, 0.0)
    ca = jnp.where(_iota(56, _L, 1) == csel, 1.0, 0.0)
    left = lax.dot_general(pa, pos_ref[bb], (((1,), (0,)), ((), ())),
                           preferred_element_type=jnp.float32,
                           precision=lax.Precision.HIGHEST)
    right = lax.dot_general(ca, cooc_ref[bb], (((1,), (0,)), ((), ())),
                            preferred_element_type=jnp.float32,
                            precision=lax.Precision.HIGHEST)
    pr = jnp.concatenate([left, right], axis=1)
    units = lax.dot_general(pr, w_ref[...], (((1,), (1,)), ((), ())),
                            preferred_element_type=jnp.float32)
    units = units + b_ref[...]
    unorm = jnp.sqrt(jnp.sum(units * units, axis=1, keepdims=True))
    units = units / (unorm + 1e-12)
    out_ref[bb] = units[0:_NUNITS]


def kernel(tokens, W, b):
    B = tokens.shape[0]
    f32 = jnp.float32

    tn = tokens / (jnp.linalg.norm(tokens, axis=-1, keepdims=True) + 1e-12)

    cooc_vec, pos_all = pl.pallas_call(
        _stage1,
        grid=(B,),
        in_specs=[
            pl.BlockSpec((1, _L, _EMBED), lambda i: (i, 0, 0)),
            pl.BlockSpec((1, _L, _EMBED), lambda i: (i, 0, 0)),
        ],
        out_specs=[
            pl.BlockSpec((1, _L, _EMBED), lambda i: (i, 0, 0)),
            pl.BlockSpec((1, _NPOS_PAD, _EMBED), lambda i: (i, 0, 0)),
        ],
        out_shape=[
            jax.ShapeDtypeStruct((B, _L, _EMBED), f32),
            jax.ShapeDtypeStruct((B, _NPOS_PAD, _EMBED), f32),
        ],
        compiler_params=pltpu.CompilerParams(
            dimension_semantics=("arbitrary",),
        ),
    )(tokens, tn)

    ncooc = cooc_vec / (jnp.linalg.norm(cooc_vec, axis=-1, keepdims=True) + 1e-12)
    npos_parts = []
    for off, n_pos in zip(_POS_OFF, _NPOS):
        seg = lax.slice_in_dim(pos_all, off, off + n_pos, axis=1)
        npos_parts.append(seg / (jnp.linalg.norm(seg, axis=-1, keepdims=True) + 1e-12))
    npos_parts.append(jnp.zeros((B, _NPOS_PAD - _NPOS_TOT, _EMBED), f32))
    npos = jnp.concatenate(npos_parts, axis=1)

    out = pl.pallas_call(
        _stage2,
        grid=(B // _BS,),
        in_specs=[
            pl.BlockSpec((_BS, _NPOS_PAD, _EMBED), lambda i: (i, 0, 0)),
            pl.BlockSpec((_BS, _L, _EMBED), lambda i: (i, 0, 0)),
            pl.BlockSpec((_BS, _NPOS_PAD, _EMBED), lambda i: (i, 0, 0)),
            pl.BlockSpec((_BS, _L, _EMBED), lambda i: (i, 0, 0)),
            pl.BlockSpec((_EMBED, 2 * _EMBED), lambda i: (0, 0)),
            pl.BlockSpec((1, _EMBED), lambda i: (0, 0)),
        ],
        out_specs=pl.BlockSpec((_BS, _NUNITS, _EMBED), lambda i: (i, 0, 0)),
        out_shape=jax.ShapeDtypeStruct((B, _NUNITS, _EMBED), f32),
        scratch_shapes=[pltpu.VMEM((_BS, _L, _L), f32)],
        compiler_params=pltpu.CompilerParams(
            dimension_semantics=("arbitrary",),
        ),
    )(npos, ncooc, pos_all, cooc_vec, W, b.reshape(1, _EMBED))
    return out
